# baseline (speedup 1.0000x reference)
import functools

import jax
import jax.numpy as jnp
from jax import lax
from jax.experimental import pallas as pl
from jax.experimental.pallas import tpu as pltpu
from jax.experimental.pallas import tpu_sc as plsc

_BETA = 0.25


def _tc_body(flat_ref, x2_ref, w_ref, w2_ref, idx_ref, loss_ref):
    b = pl.program_id(0)
    T, C = flat_ref.shape
    K = w_ref.shape[0]

    fl = flat_ref[...]
    w = w_ref[...]
    x2 = x2_ref[...]
    w2 = w2_ref[...]

    s2 = lax.dot_general(fl + fl, w, (((1,), (1,)), ((), ())),
                         preferred_element_type=jnp.float32)
    d2 = (x2 + w2) - s2

    m = jnp.min(d2, axis=1, keepdims=True)
    lanes = lax.broadcasted_iota(jnp.int32, (T, K), 1).astype(jnp.float32)
    cand = jnp.where(d2 == m, lanes, jnp.float32(K))
    idxf = jnp.min(cand, axis=1, keepdims=True)
    idx_ref[...] = idxf.astype(jnp.int32)

    part = jnp.sum(m, keepdims=True)
    @pl.when(b == 0)
    def _():
        loss_ref[...] = jnp.zeros((1, 1), jnp.float32)
    loss_ref[...] += part


def _sc_lookup(wt, idx, B, C, T, K):
    info = plsc.get_sparse_core_info()
    nc, ns, L = info.num_cores, info.num_subcores, info.num_lanes
    nw = nc * ns
    cpw = C // (nw // B)
    mesh = plsc.VectorSubcoreMesh(core_axis_name="c", subcore_axis_name="s")

    @functools.partial(
        pl.kernel, mesh=mesh,
        out_type=jax.ShapeDtypeStruct((B, C, T), jnp.float32),
        compiler_params=pltpu.CompilerParams(needs_layout_passes=False),
        scratch_types=[
            pltpu.VMEM((2 * K,), jnp.float32),
            pltpu.VMEM((T,), jnp.int32),
            pltpu.VMEM((2, T), jnp.float32),
            pltpu.SemaphoreType.DMA,
            pltpu.SemaphoreType.DMA,
        ],
    )
    def body(wt_hbm, idx_hbm, out_hbm, wt_v, idx_v, o_v, sem_in, sem_out):
        wid = lax.axis_index("s") * nc + lax.axis_index("c")
        b = wid // 2
        c0 = (wid % 2) * cpw
        pltpu.sync_copy(idx_hbm.at[b], idx_v)

        def wt_fetch(cc):
            return pltpu.async_copy(
                wt_hbm.at[pl.ds((c0 + cc) * K, K)],
                wt_v.at[pl.ds((cc % 2) * K, K)], sem_in)

        cin = [wt_fetch(0), wt_fetch(1)]
        cout = [None, None]
        for cc in range(cpw):
            slot = cc % 2
            cin[slot].wait()
            if cout[slot] is not None:
                cout[slot].wait()
            off = slot * K

            def chunk(ch, carry):
                iv = idx_v[pl.ds(ch * L, L)]
                g = plsc.load_gather(wt_v, [iv + off])
                o_v[slot, pl.ds(ch * L, L)] = g
                return carry

            lax.fori_loop(0, T // L, chunk, 0)
            cout[slot] = pltpu.async_copy(
                o_v.at[slot], out_hbm.at[b, c0 + cc], sem_out)
            if cc + 2 < cpw:
                cin[slot] = wt_fetch(cc + 2)
        cout[0].wait()
        cout[1].wait()

    return body(wt, idx)


@jax.jit
def kernel(x, W):
    B, C, T = x.shape
    K = W.shape[0]
    flat = jnp.transpose(x, (0, 2, 1)).reshape(B * T, C)
    x2 = jnp.sum(flat * flat, axis=1, keepdims=True)
    w2 = jnp.sum(W * W, axis=1)[None, :]

    idx2, loss_sum = pl.pallas_call(
        _tc_body,
        grid=(B,),
        in_specs=[
            pl.BlockSpec((T, C), lambda b: (b, 0)),
            pl.BlockSpec((T, 1), lambda b: (b, 0)),
            pl.BlockSpec((K, C), lambda b: (0, 0)),
            pl.BlockSpec((1, K), lambda b: (0, 0)),
        ],
        out_specs=[
            pl.BlockSpec((T, 1), lambda b: (b, 0)),
            pl.BlockSpec((1, 1), lambda b: (0, 0)),
        ],
        out_shape=[
            jax.ShapeDtypeStruct((B * T, 1), jnp.int32),
            jax.ShapeDtypeStruct((1, 1), jnp.float32),
        ],
    )(flat, x2, W, w2)

    idx = idx2.reshape(B, T)
    qout = _sc_lookup(W.T.reshape(-1), idx, B, C, T, K)

    codebook_loss = loss_sum[0, 0] / (B * C * T)
    commitment_loss = _BETA * codebook_loss
    return qout, codebook_loss, commitment_loss, idx


# speedup vs baseline: 2.1733x; 2.1733x over previous
import jax
import jax.numpy as jnp
from jax import lax
from jax.experimental import pallas as pl

_BETA = 0.25


def _body(x_ref, x2_ref, w_ref, qout_ref, idx_ref, cb_ref, cm_ref, acc_ref):
    b = pl.program_id(0)
    nb = pl.num_programs(0)
    C, T = x_ref.shape
    K = w_ref.shape[0]

    xb = x_ref[...]
    w = w_ref[...]
    x2 = x2_ref[...].T
    w2 = jnp.sum(w * w, axis=1, keepdims=True)

    s2 = lax.dot_general(w + w, xb, (((1,), (0,)), ((), ())),
                         preferred_element_type=jnp.float32)
    d2 = (x2 + w2) - s2

    m = jnp.min(d2, axis=0, keepdims=True)
    rows = lax.broadcasted_iota(jnp.int32, (K, T), 0).astype(jnp.float32)
    cand = jnp.where(d2 == m, rows, jnp.float32(K))
    idxf = jnp.min(cand, axis=0, keepdims=True)
    idx_ref[...] = idxf.astype(jnp.int32)

    onehot = (rows == idxf).astype(jnp.float32)
    q = lax.dot_general(w, onehot, (((0,), (0,)), ((), ())),
                        preferred_element_type=jnp.float32)
    qout_ref[...] = xb + (q - xb)

    part = jnp.sum(m, keepdims=True)[:, :1]
    @pl.when(b == 0)
    def _():
        acc_ref[...] = jnp.zeros((1, 1), jnp.float32)
    acc_ref[...] += part

    @pl.when(b == nb - 1)
    def _():
        cb = acc_ref[...] / (nb * C * T)
        cb_ref[...] = cb
        cm_ref[...] = _BETA * cb


@jax.jit
def kernel(x, W):
    B, C, T = x.shape
    K = W.shape[0]
    flat = jnp.transpose(x, (0, 2, 1)).reshape(B * T, C)
    flat = lax.optimization_barrier(flat)
    x2 = jnp.sum(flat * flat, axis=1, keepdims=True)

    qout, idx3, cb, cm, _ = pl.pallas_call(
        _body,
        grid=(B,),
        in_specs=[
            pl.BlockSpec((None, C, T), lambda b: (b, 0, 0)),
            pl.BlockSpec((T, 1), lambda b: (b, 0)),
            pl.BlockSpec((K, C), lambda b: (0, 0)),
        ],
        out_specs=[
            pl.BlockSpec((None, C, T), lambda b: (b, 0, 0)),
            pl.BlockSpec((None, 1, T), lambda b: (b, 0, 0)),
            pl.BlockSpec((1, 1), lambda b: (0, 0)),
            pl.BlockSpec((1, 1), lambda b: (0, 0)),
            pl.BlockSpec((1, 1), lambda b: (0, 0)),
        ],
        out_shape=[
            jax.ShapeDtypeStruct((B, C, T), jnp.float32),
            jax.ShapeDtypeStruct((B, 1, T), jnp.int32),
            jax.ShapeDtypeStruct((1, 1), jnp.float32),
            jax.ShapeDtypeStruct((1, 1), jnp.float32),
            jax.ShapeDtypeStruct((1, 1), jnp.float32),
        ],
    )(x, x2, W)

    return qout, cb[0, 0], cm[0, 0], idx3.reshape(B, T)


# speedup vs baseline: 2.2530x; 1.0367x over previous
import jax
import jax.numpy as jnp
from jax import lax
from jax.experimental import pallas as pl

_BETA = 0.25


def _body(x_ref, x2_ref, w_ref, qout_ref, idx_ref, cb_ref, cm_ref, acc_ref):
    b = pl.program_id(0)
    nb = pl.num_programs(0)
    C, T = x_ref.shape
    K = w_ref.shape[0]

    xb = x_ref[...]
    w = w_ref[...]
    x2 = x2_ref[...].T
    w2 = jnp.sum(w * w, axis=1, keepdims=True)

    s2 = lax.dot_general(w + w, xb, (((1,), (0,)), ((), ())),
                         preferred_element_type=jnp.float32)
    d2 = (x2 + w2) - s2

    m = jnp.min(d2, axis=0, keepdims=True)
    rows = lax.broadcasted_iota(jnp.int32, (K, T), 0).astype(jnp.float32)
    cand = jnp.where(d2 == m, rows, jnp.float32(K))
    idxf = jnp.min(cand, axis=0, keepdims=True)
    idx_ref[...] = idxf.astype(jnp.int32)

    onehot = (rows == idxf).astype(jnp.float32)
    q = lax.dot_general(w, onehot, (((0,), (0,)), ((), ())),
                        preferred_element_type=jnp.float32)
    qout_ref[...] = q

    part = jnp.sum(m, keepdims=True)[:, :1]
    @pl.when(b == 0)
    def _():
        acc_ref[...] = jnp.zeros((1, 1), jnp.float32)
    acc_ref[...] += part

    @pl.when(b == nb - 1)
    def _():
        cb = acc_ref[...] / (nb * C * T)
        cb_ref[...] = cb
        cm_ref[...] = _BETA * cb


@jax.jit
def kernel(x, W):
    B, C, T = x.shape
    K = W.shape[0]
    flat = jnp.transpose(x, (0, 2, 1)).reshape(B * T, C)
    x2 = jnp.sum(flat * flat, axis=1, keepdims=True)

    qout, idx3, cb, cm, _ = pl.pallas_call(
        _body,
        grid=(B,),
        in_specs=[
            pl.BlockSpec((None, C, T), lambda b: (b, 0, 0)),
            pl.BlockSpec((T, 1), lambda b: (b, 0)),
            pl.BlockSpec((K, C), lambda b: (0, 0)),
        ],
        out_specs=[
            pl.BlockSpec((None, C, T), lambda b: (b, 0, 0)),
            pl.BlockSpec((None, 1, T), lambda b: (b, 0, 0)),
            pl.BlockSpec((1, 1), lambda b: (0, 0)),
            pl.BlockSpec((1, 1), lambda b: (0, 0)),
            pl.BlockSpec((1, 1), lambda b: (0, 0)),
        ],
        out_shape=[
            jax.ShapeDtypeStruct((B, C, T), jnp.float32),
            jax.ShapeDtypeStruct((B, 1, T), jnp.int32),
            jax.ShapeDtypeStruct((1, 1), jnp.float32),
            jax.ShapeDtypeStruct((1, 1), jnp.float32),
            jax.ShapeDtypeStruct((1, 1), jnp.float32),
        ],
    )(x, x2, W)

    return qout, cb[0, 0], cm[0, 0], idx3.reshape(B, T)


# speedup vs baseline: 2.4956x; 1.1077x over previous
import jax
import jax.numpy as jnp
from jax import lax
from jax.experimental import pallas as pl
from jax.experimental.pallas import tpu as pltpu

_BETA = 0.25


def _body(x_ref, x2_ref, w_ref, qout_ref, idx_ref, cb_ref, cm_ref, acc_ref):
    b = pl.program_id(0)
    nb = pl.num_programs(0)
    C, T = x_ref.shape
    K = w_ref.shape[0]

    xb = x_ref[...]
    w = w_ref[...]
    x2 = x2_ref[...][None, :]
    w2 = jnp.sum(w * w, axis=1, keepdims=True)

    s2 = lax.dot_general(w + w, xb, (((1,), (0,)), ((), ())),
                         preferred_element_type=jnp.float32)
    d2 = (x2 + w2) - s2

    m = jnp.min(d2, axis=0, keepdims=True)
    rows = lax.broadcasted_iota(jnp.int32, (K, T), 0).astype(jnp.float32)
    cand = jnp.where(d2 == m, rows, jnp.float32(K))
    idxf = jnp.min(cand, axis=0, keepdims=True)
    idx_ref[...] = idxf[0].astype(jnp.int32)

    onehot = (rows == idxf).astype(jnp.float32)
    q = lax.dot_general(w, onehot, (((0,), (0,)), ((), ())),
                        preferred_element_type=jnp.float32)
    qout_ref[...] = q

    part = jnp.sum(m, keepdims=True)[:, :1]
    @pl.when(b == 0)
    def _():
        acc_ref[...] = jnp.zeros((1, 1), jnp.float32)
    acc_ref[...] += part

    @pl.when(b == nb - 1)
    def _():
        cb = acc_ref[...] / (nb * C * T)
        cb_ref[...] = cb
        cm_ref[...] = _BETA * cb


@jax.jit
def kernel(x, W):
    B, C, T = x.shape
    K = W.shape[0]
    flat = jnp.transpose(x, (0, 2, 1)).reshape(B * T, C)
    x2 = jnp.sum(flat * flat, axis=1)

    qout, idx3, cb, cm, _ = pl.pallas_call(
        _body,
        grid=(B,),
        in_specs=[
            pl.BlockSpec((None, C, T), lambda b: (b, 0, 0)),
            pl.BlockSpec((T,), lambda b: (b,)),
            pl.BlockSpec((K, C), lambda b: (0, 0)),
        ],
        out_specs=[
            pl.BlockSpec((None, C, T), lambda b: (b, 0, 0)),
            pl.BlockSpec((T,), lambda b: (b,)),
            pl.BlockSpec((1, 1), lambda b: (0, 0)),
            pl.BlockSpec((1, 1), lambda b: (0, 0)),
            pl.BlockSpec((1, 1), lambda b: (0, 0)),
        ],
        out_shape=[
            jax.ShapeDtypeStruct((B, C, T), jnp.float32),
            jax.ShapeDtypeStruct((B * T,), jnp.int32),
            jax.ShapeDtypeStruct((1, 1), jnp.float32),
            jax.ShapeDtypeStruct((1, 1), jnp.float32),
            jax.ShapeDtypeStruct((1, 1), jnp.float32),
        ],
    )(x, x2, W)

    return qout, cb[0, 0], cm[0, 0], idx3.reshape(B, T)


# speedup vs baseline: 2.4995x; 1.0016x over previous
import jax
import jax.numpy as jnp
from jax import lax
from jax.experimental import pallas as pl

_BETA = 0.25


def _body(x_ref, x2_ref, w_ref, qout_ref, idx_ref, cb_ref, cm_ref, acc_ref):
    b = pl.program_id(0)
    nb = pl.num_programs(0)
    C, T = x_ref.shape
    K = w_ref.shape[0]

    xb = x_ref[...]
    w = w_ref[...]
    x2 = x2_ref[...][None, :]
    w2 = jnp.sum(w * w, axis=1, keepdims=True)

    s2 = lax.dot_general(w + w, xb, (((1,), (0,)), ((), ())),
                         preferred_element_type=jnp.float32)
    d2 = (x2 + w2) - s2

    m = jnp.min(d2, axis=0, keepdims=True)
    rows = lax.broadcasted_iota(jnp.int32, (K, T), 0).astype(jnp.float32)
    cand = jnp.where(d2 == m, rows, jnp.float32(K))
    idxf = jnp.min(cand, axis=0, keepdims=True)
    idx_ref[...] = idxf[0].astype(jnp.int32)

    onehot = (rows == idxf).astype(jnp.float32)
    q = lax.dot_general(w, onehot, (((0,), (0,)), ((), ())),
                        preferred_element_type=jnp.float32)
    qout_ref[...] = q

    part = jnp.sum(m, keepdims=True)[:, :1]
    @pl.when(b == 0)
    def _():
        acc_ref[...] = jnp.zeros((1, 1), jnp.float32)
    acc_ref[...] += part

    @pl.when(b == nb - 1)
    def _():
        cb = acc_ref[...] / (nb * C * T)
        cb_ref[...] = cb
        cm_ref[...] = _BETA * cb


@jax.jit
def kernel(x, W):
    B, C, T = x.shape
    K = W.shape[0]
    flat = jnp.transpose(x, (0, 2, 1)).reshape(B * T, C)
    x2 = jnp.sum(flat * flat, axis=1)

    qout, idx3, cb, cm, _ = pl.pallas_call(
        _body,
        grid=(B,),
        in_specs=[
            pl.BlockSpec((None, C, T), lambda b: (b, 0, 0)),
            pl.BlockSpec((T,), lambda b: (b,)),
            pl.BlockSpec((K, C), lambda b: (0, 0)),
        ],
        out_specs=[
            pl.BlockSpec((None, C, T), lambda b: (b, 0, 0)),
            pl.BlockSpec((T,), lambda b: (b,)),
            pl.BlockSpec((1, 1), lambda b: (0, 0)),
            pl.BlockSpec((1, 1), lambda b: (0, 0)),
            pl.BlockSpec((1, 1), lambda b: (0, 0)),
        ],
        out_shape=[
            jax.ShapeDtypeStruct((B, C, T), jnp.float32),
            jax.ShapeDtypeStruct((B * T,), jnp.int32),
            jax.ShapeDtypeStruct((1, 1), jnp.float32),
            jax.ShapeDtypeStruct((1, 1), jnp.float32),
            jax.ShapeDtypeStruct((1, 1), jnp.float32),
        ],
    )(x, x2, W)

    return qout, cb[0, 0], cm[0, 0], idx3.reshape(B, T)
